# Initial kernel scaffold; baseline (speedup 1.0000x reference)
import jax
import jax.numpy as jnp
from jax.experimental import pallas as pl


def kernel(input_index_batch, output_indices_batch, W_in, W_out):
    raise NotImplementedError("write your pallas kernel here")


# speedup vs baseline: 3.6827x; 3.6827x over previous
import jax
import jax.numpy as jnp
from jax import lax
from jax.experimental import pallas as pl
from jax.experimental.pallas import tpu as pltpu
from jax.experimental.pallas import tpu_sc as plsc

D = 256
K = 21
B = 16384
NC, NS = 2, 16
NW = NC * NS
BPW = B // NW
C = 16
CH = BPW // C
KC = C * K
IDXW = 112
NG = KC // IDXW
LANES = 16


def _sc_body(in_idx_hbm, oidx_hbm, w_inT_hbm, w_out_hbm, out_hbm,
             uidx_v, vidx_v, u_rows, v_rows, out_buf, sem):
    wid = lax.axis_index("s") * NC + lax.axis_index("c")
    base = wid * BPW
    lane = lax.broadcasted_iota(jnp.int32, (LANES,), 0)
    row_idx = [lane * K + k for k in range(K)]

    def chunk(j, carry):
        sb = base + j * C
        pltpu.sync_copy(in_idx_hbm.at[pl.ds(sb, C)], uidx_v)
        pltpu.sync_copy(oidx_hbm.at[pl.ds(sb * K, KC)], vidx_v)
        cp_u = pltpu.async_copy(w_inT_hbm.at[uidx_v], u_rows, sem)
        cps = [pltpu.async_copy(w_out_hbm.at[vidx_v.at[pl.ds(g * IDXW, IDXW)]],
                                v_rows.at[pl.ds(g * IDXW, IDXW)], sem)
               for g in range(NG)]
        cp_u.wait()
        for cp in cps:
            cp.wait()

        def depth(d, accs):
            dvec = jnp.full((LANES,), d, jnp.int32)
            uvec = plsc.load_gather(u_rows, [lane, dvec])
            return tuple(
                accs[k] + plsc.load_gather(v_rows, [row_idx[k], dvec]) * uvec
                for k in range(K))

        accs = lax.fori_loop(
            0, D, depth, tuple(jnp.zeros((LANES,), jnp.float32)
                               for _ in range(K)))
        for k in range(K):
            plsc.store_scatter(out_buf, [row_idx[k]], accs[k])
        pltpu.sync_copy(out_buf, out_hbm.at[pl.ds(sb * K, KC)])
        return carry

    lax.fori_loop(0, CH, chunk, 0)


def _sc_predictions(in_idx, oidx2d, w_inT, w_out, *, interpret=False):
    mesh = plsc.VectorSubcoreMesh(core_axis_name="c", subcore_axis_name="s",
                                  num_cores=NC, num_subcores=NS)
    f = pl.kernel(
        _sc_body,
        out_type=jax.ShapeDtypeStruct((B * K,), jnp.float32),
        mesh=mesh,
        scratch_types=[
            pltpu.VMEM((C,), jnp.int32),
            pltpu.VMEM((KC,), jnp.int32),
            pltpu.VMEM((C, D), jnp.float32),
            pltpu.VMEM((KC, D), jnp.float32),
            pltpu.VMEM((KC,), jnp.float32),
            pltpu.SemaphoreType.DMA,
        ],
        compiler_params=pltpu.CompilerParams(needs_layout_passes=False),
        interpret=interpret,
    )
    return f(in_idx, oidx2d, w_inT, w_out)


@jax.jit
def kernel(input_index_batch, output_indices_batch, W_in, W_out):
    in_idx = input_index_batch.astype(jnp.int32)
    oidx = output_indices_batch.astype(jnp.int32).reshape(B * K)
    w_inT = W_in.T
    flat = _sc_predictions(in_idx, oidx, w_inT, W_out)
    return flat.reshape(B, K)


# speedup vs baseline: 14.7630x; 4.0088x over previous
import jax
import jax.numpy as jnp
from jax import lax
from jax.experimental import pallas as pl
from jax.experimental.pallas import tpu as pltpu
from jax.experimental.pallas import tpu_sc as plsc

D = 256
K = 21
B = 16384
NC, NS = 2, 16
NW = NC * NS
BPW = B // NW
C = 16
CH = BPW // C
KC = C * K
IDXW = 112
NG = KC // IDXW
DT = D // 16


def _sc_body(in_idx_hbm, oidx_hbm, w_inT_hbm, w_out_hbm, out_hbm,
             uidx_v, vidx_v, u_rows, v_rows, out_buf, sem):
    wid = lax.axis_index("s") * NC + lax.axis_index("c")
    base = wid * BPW

    def chunk(j, carry):
        sb = base + j * C
        pltpu.sync_copy(in_idx_hbm.at[pl.ds(sb, C)], uidx_v)
        pltpu.sync_copy(oidx_hbm.at[pl.ds(sb * K, KC)], vidx_v)
        cp_u = pltpu.async_copy(w_inT_hbm.at[uidx_v], u_rows, sem)
        cps = [pltpu.async_copy(w_out_hbm.at[vidx_v.at[pl.ds(g * IDXW, IDXW)]],
                                v_rows.at[pl.ds(g * IDXW, IDXW)], sem)
               for g in range(NG)]
        cp_u.wait()
        for cp in cps:
            cp.wait()

        last_lane = lax.broadcasted_iota(jnp.int32, (16,), 0) == 15

        def sample(s, carry2):
            uvecs = [u_rows[s, pl.ds(16 * t, 16)] for t in range(DT)]

            def kdot(k, carry3):
                r = s * K + k
                accs = [v_rows[r, pl.ds(16 * t, 16)] * uvecs[t]
                        for t in range(4)]
                for t in range(4, DT):
                    accs[t % 4] = accs[t % 4] + (
                        v_rows[r, pl.ds(16 * t, 16)] * uvecs[t])
                tot = plsc.cumsum((accs[0] + accs[1]) + (accs[2] + accs[3]))
                plsc.store_scatter(out_buf, [jnp.full((16,), r, jnp.int32)],
                                   tot, mask=last_lane)
                return carry3

            lax.fori_loop(0, K, kdot, 0)
            return carry2

        lax.fori_loop(0, C, sample, 0)
        pltpu.sync_copy(out_buf, out_hbm.at[pl.ds(sb * K, KC)])
        return carry

    lax.fori_loop(0, CH, chunk, 0)


def _sc_predictions(in_idx, oidx, w_inT, w_out):
    mesh = plsc.VectorSubcoreMesh(core_axis_name="c", subcore_axis_name="s",
                                  num_cores=NC, num_subcores=NS)
    f = pl.kernel(
        _sc_body,
        out_type=jax.ShapeDtypeStruct((B * K,), jnp.float32),
        mesh=mesh,
        scratch_types=[
            pltpu.VMEM((C,), jnp.int32),
            pltpu.VMEM((KC,), jnp.int32),
            pltpu.VMEM((C, D), jnp.float32),
            pltpu.VMEM((KC, D), jnp.float32),
            pltpu.VMEM((KC,), jnp.float32),
            pltpu.SemaphoreType.DMA,
        ],
        compiler_params=pltpu.CompilerParams(needs_layout_passes=False),
    )
    return f(in_idx, oidx, w_inT, w_out)


@jax.jit
def kernel(input_index_batch, output_indices_batch, W_in, W_out):
    in_idx = input_index_batch.astype(jnp.int32)
    oidx = output_indices_batch.astype(jnp.int32).reshape(B * K)
    w_inT = W_in.T
    flat = _sc_predictions(in_idx, oidx, w_inT, W_out)
    return flat.reshape(B, K)


# speedup vs baseline: 18.2774x; 1.2381x over previous
import jax
import jax.numpy as jnp
from jax import lax
from jax.experimental import pallas as pl
from jax.experimental.pallas import tpu as pltpu
from jax.experimental.pallas import tpu_sc as plsc

D = 256
K = 21
B = 16384
NC, NS = 2, 16
NW = NC * NS
BPW = B // NW
C = 8
CH = BPW // C
KC = C * K
IDXW = 56
NG = KC // IDXW
DT = D // 16


def _sc_body(in_idx_hbm, oidx_hbm, w_inT_hbm, w_out_hbm, out_hbm,
             uidx_a, vidx_a, u_a, v_a, out_a, sem_a,
             uidx_b, vidx_b, u_b, v_b, out_b, sem_b):
    wid = lax.axis_index("s") * NC + lax.axis_index("c")
    base = wid * BPW
    last_lane = lax.broadcasted_iota(jnp.int32, (16,), 0) == 15

    def issue(ci, uidx, vidx, u_rows, v_rows, sem):
        sb = base + ci * C
        pltpu.sync_copy(in_idx_hbm.at[pl.ds(sb, C)], uidx)
        pltpu.sync_copy(oidx_hbm.at[pl.ds(sb * K, KC)], vidx)
        pltpu.async_copy(w_inT_hbm.at[uidx], u_rows, sem)
        for g in range(NG):
            pltpu.async_copy(w_out_hbm.at[vidx.at[pl.ds(g * IDXW, IDXW)]],
                             v_rows.at[pl.ds(g * IDXW, IDXW)], sem)

    def drain(uidx, vidx, u_rows, v_rows, sem):
        pltpu.make_async_copy(w_inT_hbm.at[uidx], u_rows, sem).wait()
        for g in range(NG):
            pltpu.make_async_copy(w_out_hbm.at[vidx.at[pl.ds(g * IDXW, IDXW)]],
                                  v_rows.at[pl.ds(g * IDXW, IDXW)], sem).wait()

    def compute(ci, u_rows, v_rows, out_buf):
        def sample(s, carry2):
            uvecs = [u_rows[s, pl.ds(16 * t, 16)] for t in range(DT)]

            def kdot(k, carry3):
                r = s * K + k
                accs = [v_rows[r, pl.ds(16 * t, 16)] * uvecs[t]
                        for t in range(4)]
                for t in range(4, DT):
                    accs[t % 4] = accs[t % 4] + (
                        v_rows[r, pl.ds(16 * t, 16)] * uvecs[t])
                tot = plsc.cumsum((accs[0] + accs[1]) + (accs[2] + accs[3]))
                plsc.store_scatter(out_buf, [jnp.full((16,), r, jnp.int32)],
                                   tot, mask=last_lane)
                return carry3

            lax.fori_loop(0, K, kdot, 0)
            return carry2

        lax.fori_loop(0, C, sample, 0)
        sb = base + ci * C
        pltpu.sync_copy(out_buf, out_hbm.at[pl.ds(sb * K, KC)])

    bufs_a = (uidx_a, vidx_a, u_a, v_a, sem_a)
    bufs_b = (uidx_b, vidx_b, u_b, v_b, sem_b)

    issue(0, *bufs_a)
    issue(1, *bufs_b)

    def pair(t, carry):
        ca = 2 * t
        drain(*bufs_a)
        compute(ca, u_a, v_a, out_a)
        issue(jnp.minimum(ca + 2, CH - 1), *bufs_a)
        drain(*bufs_b)
        compute(ca + 1, u_b, v_b, out_b)
        issue(jnp.minimum(ca + 3, CH - 1), *bufs_b)
        return carry

    lax.fori_loop(0, CH // 2, pair, 0)
    drain(*bufs_a)
    drain(*bufs_b)


def _sc_predictions(in_idx, oidx, w_inT, w_out):
    mesh = plsc.VectorSubcoreMesh(core_axis_name="c", subcore_axis_name="s",
                                  num_cores=NC, num_subcores=NS)
    buf_set = [
        pltpu.VMEM((C,), jnp.int32),
        pltpu.VMEM((KC,), jnp.int32),
        pltpu.VMEM((C, D), jnp.float32),
        pltpu.VMEM((KC, D), jnp.float32),
        pltpu.VMEM((KC,), jnp.float32),
        pltpu.SemaphoreType.DMA,
    ]
    f = pl.kernel(
        _sc_body,
        out_type=jax.ShapeDtypeStruct((B * K,), jnp.float32),
        mesh=mesh,
        scratch_types=buf_set + buf_set,
        compiler_params=pltpu.CompilerParams(needs_layout_passes=False),
    )
    return f(in_idx, oidx, w_inT, w_out)


@jax.jit
def kernel(input_index_batch, output_indices_batch, W_in, W_out):
    in_idx = input_index_batch.astype(jnp.int32)
    oidx = output_indices_batch.astype(jnp.int32).reshape(B * K)
    w_inT = W_in.T
    flat = _sc_predictions(in_idx, oidx, w_inT, w_out=W_out)
    return flat.reshape(B, K)


# speedup vs baseline: 21.4618x; 1.1742x over previous
import jax
import jax.numpy as jnp
from jax import lax
from jax.experimental import pallas as pl
from jax.experimental.pallas import tpu as pltpu
from jax.experimental.pallas import tpu_sc as plsc

D = 256
K = 21
B = 16384
NC, NS = 2, 16
NW = NC * NS
BPW = B // NW
C = 8
CH = BPW // C
KC = C * K
IDXW = 56
NG = KC // IDXW
DT = D // 16


def _sc_body(in_idx_hbm, oidx_hbm, w_inT_hbm, w_out_hbm, out_hbm,
             uidx_all, vidx_all,
             u_a, v_a, out_a, sem_a,
             u_b, v_b, out_b, sem_b):
    wid = lax.axis_index("s") * NC + lax.axis_index("c")
    base = wid * BPW
    last_lane = lax.broadcasted_iota(jnp.int32, (16,), 0) == 15

    pltpu.sync_copy(in_idx_hbm.at[pl.ds(base, BPW)], uidx_all)
    pltpu.sync_copy(oidx_hbm.at[pl.ds(base * K, BPW * K)], vidx_all)

    def issue(ci, u_rows, v_rows, sem):
        pltpu.async_copy(w_inT_hbm.at[uidx_all.at[pl.ds(ci * C, C)]],
                         u_rows, sem)
        for g in range(NG):
            pltpu.async_copy(
                w_out_hbm.at[vidx_all.at[pl.ds(ci * KC + g * IDXW, IDXW)]],
                v_rows.at[pl.ds(g * IDXW, IDXW)], sem)

    def drain(ci, u_rows, v_rows, sem):
        pltpu.make_async_copy(w_inT_hbm.at[uidx_all.at[pl.ds(ci * C, C)]],
                              u_rows, sem).wait()
        for g in range(NG):
            pltpu.make_async_copy(
                w_out_hbm.at[vidx_all.at[pl.ds(ci * KC + g * IDXW, IDXW)]],
                v_rows.at[pl.ds(g * IDXW, IDXW)], sem).wait()

    def compute(ci, u_rows, v_rows, out_buf):
        def sample(s, carry2):
            uvecs = [u_rows[s, pl.ds(16 * t, 16)] for t in range(DT)]

            def kdot(k, carry3):
                r = s * K + k
                accs = [v_rows[r, pl.ds(16 * t, 16)] * uvecs[t]
                        for t in range(4)]
                for t in range(4, DT):
                    accs[t % 4] = accs[t % 4] + (
                        v_rows[r, pl.ds(16 * t, 16)] * uvecs[t])
                tot = plsc.cumsum((accs[0] + accs[1]) + (accs[2] + accs[3]))
                plsc.store_scatter(out_buf, [jnp.full((16,), r, jnp.int32)],
                                   tot, mask=last_lane)
                return carry3

            lax.fori_loop(0, K, kdot, 0)
            return carry2

        lax.fori_loop(0, C, sample, 0)
        sb = base + ci * C
        pltpu.sync_copy(out_buf, out_hbm.at[pl.ds(sb * K, KC)])

    issue(0, u_a, v_a, sem_a)
    issue(1, u_b, v_b, sem_b)

    def pair(t, carry):
        ca = 2 * t
        drain(ca, u_a, v_a, sem_a)
        compute(ca, u_a, v_a, out_a)
        issue(jnp.minimum(ca + 2, CH - 1), u_a, v_a, sem_a)
        drain(ca + 1, u_b, v_b, sem_b)
        compute(ca + 1, u_b, v_b, out_b)
        issue(jnp.minimum(ca + 3, CH - 1), u_b, v_b, sem_b)
        return carry

    lax.fori_loop(0, CH // 2, pair, 0)
    drain(CH - 1, u_a, v_a, sem_a)
    drain(CH - 1, u_b, v_b, sem_b)


def _sc_predictions(in_idx, oidx, w_inT, w_out):
    mesh = plsc.VectorSubcoreMesh(core_axis_name="c", subcore_axis_name="s",
                                  num_cores=NC, num_subcores=NS)
    idx_set = [
        pltpu.VMEM((BPW,), jnp.int32),
        pltpu.VMEM((BPW * K,), jnp.int32),
    ]
    buf_set = [
        pltpu.VMEM((C, D), jnp.float32),
        pltpu.VMEM((KC, D), jnp.float32),
        pltpu.VMEM((KC,), jnp.float32),
        pltpu.SemaphoreType.DMA,
    ]
    f = pl.kernel(
        _sc_body,
        out_type=jax.ShapeDtypeStruct((B * K,), jnp.float32),
        mesh=mesh,
        scratch_types=idx_set + buf_set + buf_set,
        compiler_params=pltpu.CompilerParams(needs_layout_passes=False),
    )
    return f(in_idx, oidx, w_inT, w_out)


@jax.jit
def kernel(input_index_batch, output_indices_batch, W_in, W_out):
    in_idx = input_index_batch.astype(jnp.int32)
    oidx = output_indices_batch.astype(jnp.int32).reshape(B * K)
    w_inT = W_in.T
    flat = _sc_predictions(in_idx, oidx, w_inT, w_out=W_out)
    return flat.reshape(B, K)
